# baseline (speedup 1.0000x reference)
import functools

import jax
import jax.numpy as jnp
from jax import lax
from jax.experimental import pallas as pl
from jax.experimental.pallas import tpu as pltpu
from jax.experimental.pallas import tpu_sc as plsc


def kernel(x, token_table, pos_table):
    B, M = x.shape
    V, D = token_table.shape
    NC, NS = 2, 16
    NW = NC * NS
    R = B * M
    per_w = R // NW
    CH = 2 * M
    NCHUNK = per_w // CH
    G = 100
    NG = CH // G
    NBUF = 4
    NSTEP = NCHUNK // NBUF

    assert per_w * NW == R and NCHUNK * CH == per_w and NG * G == CH
    assert NSTEP * NBUF == NCHUNK

    x_flat = x.astype(jnp.int32).reshape(NW * NCHUNK, NG, G)

    mesh = plsc.VectorSubcoreMesh(core_axis_name="c", subcore_axis_name="s")

    @functools.partial(
        pl.kernel,
        mesh=mesh,
        compiler_params=pltpu.CompilerParams(use_tc_tiling_on_sc=False),
        out_type=jax.ShapeDtypeStruct((R, D), jnp.float32),
        scratch_types=[
            pltpu.VMEM((NBUF, NG, G), jnp.int32),
            pltpu.VMEM((NBUF, CH, D), jnp.float32),
            pltpu.VMEM((M, D), jnp.float32),
            pltpu.SemaphoreType.DMA,
            pltpu.SemaphoreType.DMA,
            pltpu.SemaphoreType.DMA,
            pltpu.SemaphoreType.DMA,
            pltpu.SemaphoreType.DMA,
            pltpu.SemaphoreType.DMA,
            pltpu.SemaphoreType.DMA,
            pltpu.SemaphoreType.DMA,
            pltpu.SemaphoreType.DMA,
            pltpu.SemaphoreType.DMA,
            pltpu.SemaphoreType.DMA,
            pltpu.SemaphoreType.DMA,
        ],
    )
    def sc_kernel(x_hbm, tok_hbm, pos_hbm, out_hbm, idx_v, rows_v, pos_v,
                  g0, g1, g2, g3, w0, w1, w2, w3, i0, i1, i2, i3):
        wid = lax.axis_index("s") * NC + lax.axis_index("c")
        gsem = [g0, g1, g2, g3]
        wsem = [w0, w1, w2, w3]
        isem = [i0, i1, i2, i3]
        pltpu.sync_copy(pos_hbm, pos_v)

        def fire_idx(c, nb):
            pltpu.async_copy(x_hbm.at[wid * NCHUNK + c], idx_v.at[nb], isem[nb])

        def wait_idx(c, nb):
            pltpu.make_async_copy(
                x_hbm.at[wid * NCHUNK + c], idx_v.at[nb], isem[nb]
            ).wait()

        def fire_gathers(nb):
            for g in range(NG):
                pltpu.async_copy(
                    tok_hbm.at[idx_v.at[nb, g]],
                    rows_v.at[nb].at[pl.ds(g * G, G)],
                    gsem[nb],
                )

        def wait_gathers(nb):
            for g in range(NG):
                pltpu.make_async_copy(
                    tok_hbm.at[idx_v.at[nb, g]],
                    rows_v.at[nb].at[pl.ds(g * G, G)],
                    gsem[nb],
                ).wait()

        def add_pos(nb):
            rb = rows_v.at[nb]

            def add_body(m, carry):
                p0 = pos_v[m, pl.ds(0, 16)]
                p1 = pos_v[m, pl.ds(16, 16)]
                for rep in range(CH // M):
                    r = rep * M + m
                    rb[r, pl.ds(0, 16)] = rb[r, pl.ds(0, 16)] + p0
                    rb[r, pl.ds(16, 16)] = rb[r, pl.ds(16, 16)] + p1
                return carry

            lax.fori_loop(0, M, add_body, 0)

        def fire_writeout(c, nb):
            pltpu.async_copy(
                rows_v.at[nb],
                out_hbm.at[pl.ds((wid * NCHUNK + c) * CH, CH)],
                wsem[nb],
            )

        def wait_writeout(c, nb):
            pltpu.make_async_copy(
                rows_v.at[nb],
                out_hbm.at[pl.ds((wid * NCHUNK + c) * CH, CH)],
                wsem[nb],
            ).wait()

        fire_idx(0, 0)
        fire_idx(1, 1)
        wait_idx(0, 0)
        fire_gathers(0)

        def outer(i, carry):
            for j in range(NBUF):
                c = NBUF * i + j

                jb2 = (j + 2) % NBUF
                if j < 2:
                    fire_idx(c + 2, jb2)
                else:
                    @pl.when(i < NSTEP - 1)
                    def _():
                        fire_idx(c + 2, jb2)

                jb1 = (j + 1) % NBUF
                if j == NBUF - 1:
                    wait_writeout(c - 3, jb1)
                else:
                    @pl.when(i > 0)
                    def _():
                        wait_writeout(c - 3, jb1)

                if j == NBUF - 1:
                    @pl.when(i < NSTEP - 1)
                    def _():
                        wait_idx(c + 1, jb1)
                        fire_gathers(jb1)
                else:
                    wait_idx(c + 1, jb1)
                    fire_gathers(jb1)

                wait_gathers(j)
                add_pos(j)
                fire_writeout(c, j)
            return carry

        lax.fori_loop(0, NSTEP, outer, 0)
        wait_writeout(NCHUNK - 3, 1)
        wait_writeout(NCHUNK - 2, 2)
        wait_writeout(NCHUNK - 1, 3)

    out = sc_kernel(x_flat, token_table, pos_table)
    return out.reshape(B, M, D)


# speedup vs baseline: 1.0092x; 1.0092x over previous
import functools

import jax
import jax.numpy as jnp
from jax import lax
from jax.experimental import pallas as pl
from jax.experimental.pallas import tpu as pltpu
from jax.experimental.pallas import tpu_sc as plsc


def kernel(x, token_table, pos_table):
    B, M = x.shape
    V, D = token_table.shape
    NC, NS = 2, 16
    NW = NC * NS
    R = B * M
    per_w = R // NW
    CH = 4 * M
    NCHUNK = per_w // CH
    G = 100
    NG = CH // G
    NBUF = 4
    NSTEP = NCHUNK // NBUF

    assert per_w * NW == R and NCHUNK * CH == per_w and NG * G == CH
    assert NSTEP * NBUF == NCHUNK

    x_flat = x.astype(jnp.int32).reshape(NW * NCHUNK, NG, G)

    mesh = plsc.VectorSubcoreMesh(core_axis_name="c", subcore_axis_name="s")

    @functools.partial(
        pl.kernel,
        mesh=mesh,
        compiler_params=pltpu.CompilerParams(use_tc_tiling_on_sc=False),
        out_type=jax.ShapeDtypeStruct((R, D), jnp.float32),
        scratch_types=[
            pltpu.VMEM((NBUF, NG, G), jnp.int32),
            pltpu.VMEM((NBUF, CH, D), jnp.float32),
            pltpu.VMEM((M, D), jnp.float32),
            pltpu.SemaphoreType.DMA,
            pltpu.SemaphoreType.DMA,
            pltpu.SemaphoreType.DMA,
            pltpu.SemaphoreType.DMA,
            pltpu.SemaphoreType.DMA,
            pltpu.SemaphoreType.DMA,
            pltpu.SemaphoreType.DMA,
            pltpu.SemaphoreType.DMA,
            pltpu.SemaphoreType.DMA,
            pltpu.SemaphoreType.DMA,
            pltpu.SemaphoreType.DMA,
            pltpu.SemaphoreType.DMA,
        ],
    )
    def sc_kernel(x_hbm, tok_hbm, pos_hbm, out_hbm, idx_v, rows_v, pos_v,
                  g0, g1, g2, g3, w0, w1, w2, w3, i0, i1, i2, i3):
        wid = lax.axis_index("s") * NC + lax.axis_index("c")
        gsem = [g0, g1, g2, g3]
        wsem = [w0, w1, w2, w3]
        isem = [i0, i1, i2, i3]
        pltpu.sync_copy(pos_hbm, pos_v)

        def fire_idx(c, nb):
            pltpu.async_copy(x_hbm.at[wid * NCHUNK + c], idx_v.at[nb], isem[nb])

        def wait_idx(c, nb):
            pltpu.make_async_copy(
                x_hbm.at[wid * NCHUNK + c], idx_v.at[nb], isem[nb]
            ).wait()

        def fire_gathers(nb):
            for g in range(NG):
                pltpu.async_copy(
                    tok_hbm.at[idx_v.at[nb, g]],
                    rows_v.at[nb].at[pl.ds(g * G, G)],
                    gsem[nb],
                )

        def wait_gathers(nb):
            for g in range(NG):
                pltpu.make_async_copy(
                    tok_hbm.at[idx_v.at[nb, g]],
                    rows_v.at[nb].at[pl.ds(g * G, G)],
                    gsem[nb],
                ).wait()

        def add_pos(nb):
            rb = rows_v.at[nb]

            def add_body(m, carry):
                p0 = pos_v[m, pl.ds(0, 16)]
                p1 = pos_v[m, pl.ds(16, 16)]
                for rep in range(CH // M):
                    r = rep * M + m
                    rb[r, pl.ds(0, 16)] = rb[r, pl.ds(0, 16)] + p0
                    rb[r, pl.ds(16, 16)] = rb[r, pl.ds(16, 16)] + p1
                return carry

            lax.fori_loop(0, M, add_body, 0)

        def fire_writeout(c, nb):
            pltpu.async_copy(
                rows_v.at[nb],
                out_hbm.at[pl.ds((wid * NCHUNK + c) * CH, CH)],
                wsem[nb],
            )

        def wait_writeout(c, nb):
            pltpu.make_async_copy(
                rows_v.at[nb],
                out_hbm.at[pl.ds((wid * NCHUNK + c) * CH, CH)],
                wsem[nb],
            ).wait()

        fire_idx(0, 0)
        fire_idx(1, 1)
        wait_idx(0, 0)
        fire_gathers(0)

        def outer(i, carry):
            for j in range(NBUF):
                c = NBUF * i + j

                jb2 = (j + 2) % NBUF
                if j < 2:
                    fire_idx(c + 2, jb2)
                else:
                    @pl.when(i < NSTEP - 1)
                    def _():
                        fire_idx(c + 2, jb2)

                jb1 = (j + 1) % NBUF
                if j == NBUF - 1:
                    wait_writeout(c - 3, jb1)
                else:
                    @pl.when(i > 0)
                    def _():
                        wait_writeout(c - 3, jb1)

                if j == NBUF - 1:
                    @pl.when(i < NSTEP - 1)
                    def _():
                        wait_idx(c + 1, jb1)
                        fire_gathers(jb1)
                else:
                    wait_idx(c + 1, jb1)
                    fire_gathers(jb1)

                wait_gathers(j)
                add_pos(j)
                fire_writeout(c, j)
            return carry

        lax.fori_loop(0, NSTEP, outer, 0)
        wait_writeout(NCHUNK - 3, 1)
        wait_writeout(NCHUNK - 2, 2)
        wait_writeout(NCHUNK - 1, 3)

    out = sc_kernel(x_flat, token_table, pos_table)
    return out.reshape(B, M, D)
